# Initial kernel scaffold; baseline (speedup 1.0000x reference)
import functools

import jax
import jax.numpy as jnp
from jax import lax
from jax.experimental import pallas as pl
from jax.experimental.pallas import tpu as pltpu
from jax.experimental.pallas import tpu_sc as plsc

N = 10000
E = 320000
D_FEAT = 128
D_H = 64
D_LATENT = 32
DW = 16

NC, NS = 2, 16
NW = NC * NS
EPW = E // NW
CH = 80
NCH = EPW // CH

RPT_DEG = 2 * N // NS
RPT = N // NS

_MESH = plsc.VectorSubcoreMesh(core_axis_name="c", subcore_axis_name="s")


@functools.partial(
    pl.kernel,
    out_type=jax.ShapeDtypeStruct((NC, 2 * N, DW), jnp.float32),
    mesh=_MESH,
    scratch_types=[
        pltpu.VMEM((NCH, CH), jnp.int32),
        pltpu.VMEM((NCH, CH), jnp.int32),
        pltpu.VMEM((CH, DW), jnp.float32),
        pltpu.VMEM_SHARED((2 * N, DW), jnp.float32),
    ],
)
def _deg_kernel(src_h, dstn_h, ones_h, zeros_h, out_h, srcv, dstv, onesv, acc):
    c = lax.axis_index("c")
    s = lax.axis_index("s")
    wid = s * NC + c
    pltpu.sync_copy(src_h.at[wid], srcv)
    pltpu.sync_copy(dstn_h.at[wid], dstv)
    pltpu.sync_copy(ones_h, onesv)
    pltpu.sync_copy(zeros_h.at[pl.ds(s * RPT_DEG, RPT_DEG)],
                    acc.at[pl.ds(s * RPT_DEG, RPT_DEG)])
    plsc.subcore_barrier()

    def body(j, carry):
        pltpu.sync_copy(onesv, acc.at[srcv.at[j]], add=True)
        pltpu.sync_copy(onesv, acc.at[dstv.at[j]], add=True)
        return carry

    lax.fori_loop(0, NCH, body, 0)
    plsc.subcore_barrier()
    pltpu.sync_copy(acc.at[pl.ds(s * RPT_DEG, RPT_DEG)],
                    out_h.at[c, pl.ds(s * RPT_DEG, RPT_DEG)])


@functools.partial(
    pl.kernel,
    out_type=jax.ShapeDtypeStruct((NC, N, D_H), jnp.float32),
    mesh=_MESH,
    scratch_types=[
        pltpu.VMEM((EPW,), jnp.int32),
        pltpu.VMEM((NCH, CH), jnp.int32),
        pltpu.VMEM((CH, D_H), jnp.float32),
        pltpu.VMEM_SHARED((N, D_H), jnp.float32),
        pltpu.SemaphoreType.DMA,
    ],
)
def _agg_kernel(src_h, dst_h, table_h, zeros_h, out_h, srcv, dstv, rows, acc, sem):
    c = lax.axis_index("c")
    s = lax.axis_index("s")
    wid = s * NC + c
    pltpu.sync_copy(src_h.at[wid], srcv)
    pltpu.sync_copy(dst_h.at[wid], dstv)
    pltpu.sync_copy(zeros_h.at[pl.ds(s * RPT, RPT)], acc.at[pl.ds(s * RPT, RPT)])
    plsc.subcore_barrier()

    def body(j, carry):
        off = pl.multiple_of(j * CH, 8)
        pltpu.async_copy(table_h.at[srcv.at[pl.ds(off, CH)]], rows, sem).wait()
        pltpu.sync_copy(rows, acc.at[dstv.at[j]], add=True)
        return carry

    lax.fori_loop(0, NCH, body, 0)
    plsc.subcore_barrier()
    pltpu.sync_copy(acc.at[pl.ds(s * RPT, RPT)], out_h.at[c, pl.ds(s * RPT, RPT)])


BR = 1000
GRID = N // BR


def _tc_a_body(degp_ref, x_ref, w_ref, t_ref, scales_ref):
    dp = degp_ref[...]
    deg = jnp.maximum(jnp.sum(dp, axis=(0, 3)), 1.0)
    sc = lax.rsqrt(deg)
    scales_ref[...] = sc
    t_ref[...] = jnp.dot(x_ref[...], w_ref[...],
                         preferred_element_type=jnp.float32) * sc[0][:, None]


def _tc_a(degp, x, W1):
    return pl.pallas_call(
        _tc_a_body,
        grid=(GRID,),
        in_specs=[
            pl.BlockSpec((NC, 2, BR, DW), lambda i: (0, 0, i, 0)),
            pl.BlockSpec((BR, D_FEAT), lambda i: (i, 0)),
            pl.BlockSpec((D_FEAT, D_H), lambda i: (0, 0)),
        ],
        out_specs=[
            pl.BlockSpec((BR, D_H), lambda i: (i, 0)),
            pl.BlockSpec((2, BR), lambda i: (0, i)),
        ],
        out_shape=[
            jax.ShapeDtypeStruct((N, D_H), jnp.float32),
            jax.ShapeDtypeStruct((2, N), jnp.float32),
        ],
    )(degp, x, W1)


def _tc_b_body(aggp_ref, scales_ref, w_ref, t_ref):
    sc = scales_ref[...]
    a = aggp_ref[...]
    h = jnp.tanh((a[0] + a[1]) * sc[1][:, None])
    t_ref[...] = jnp.dot(h, w_ref[...],
                         preferred_element_type=jnp.float32) * sc[0][:, None]


def _tc_b(aggp, scales, W):
    return pl.pallas_call(
        _tc_b_body,
        grid=(GRID,),
        in_specs=[
            pl.BlockSpec((NC, BR, D_H), lambda i: (0, i, 0)),
            pl.BlockSpec((2, BR), lambda i: (0, i)),
            pl.BlockSpec((D_H, D_H), lambda i: (0, 0)),
        ],
        out_specs=pl.BlockSpec((BR, D_H), lambda i: (i, 0)),
        out_shape=jax.ShapeDtypeStruct((N, D_H), jnp.float32),
    )(aggp, scales, W)


def _tc_c_body(aggp_ref, scales_ref, eps_ref, z_ref, m_ref, s_ref):
    sc = scales_ref[...]
    a = aggp_ref[...]
    ms = (a[0] + a[1]) * sc[1][:, None]
    m = ms[:, :D_LATENT]
    std = jnp.maximum(ms[:, D_LATENT:], 0.0) + 0.0001
    m_ref[...] = m
    s_ref[...] = std
    z_ref[...] = eps_ref[...] * std + m


def _tc_c(aggp, scales, eps):
    return pl.pallas_call(
        _tc_c_body,
        grid=(GRID,),
        in_specs=[
            pl.BlockSpec((NC, BR, D_H), lambda i: (0, i, 0)),
            pl.BlockSpec((2, BR), lambda i: (0, i)),
            pl.BlockSpec((BR, D_LATENT), lambda i: (i, 0)),
        ],
        out_specs=[
            pl.BlockSpec((BR, D_LATENT), lambda i: (i, 0)),
            pl.BlockSpec((BR, D_LATENT), lambda i: (i, 0)),
            pl.BlockSpec((BR, D_LATENT), lambda i: (i, 0)),
        ],
        out_shape=[
            jax.ShapeDtypeStruct((N, D_LATENT), jnp.float32),
            jax.ShapeDtypeStruct((N, D_LATENT), jnp.float32),
            jax.ShapeDtypeStruct((N, D_LATENT), jnp.float32),
        ],
    )(aggp, scales, eps)


def kernel(adj, x, W1, W2, Wm, Ws):
    src = adj[0].astype(jnp.int32)
    dst = adj[1].astype(jnp.int32)
    src2 = src.reshape(NW, EPW)
    src3 = src.reshape(NW, NCH, CH)
    dst3 = dst.reshape(NW, NCH, CH)
    dstn3 = (dst + N).reshape(NW, NCH, CH)
    ones_rows = jnp.ones((CH, DW), jnp.float32)
    zeros_deg = jnp.zeros((2 * N, DW), jnp.float32)
    zeros_acc = jnp.zeros((N, D_H), jnp.float32)

    degp = _deg_kernel(src3, dstn3, ones_rows, zeros_deg)
    degp = degp.reshape(NC, 2, N, DW)
    t1, scales = _tc_a(degp, x, W1)

    a1 = _agg_kernel(src2, dst3, t1, zeros_acc)
    t2 = _tc_b(a1, scales, W2)

    a2 = _agg_kernel(src2, dst3, t2, zeros_acc)
    W34 = jnp.concatenate([Wm, Ws], axis=1)
    t3 = _tc_b(a2, scales, W34)

    a3 = _agg_kernel(src2, dst3, t3, zeros_acc)
    eps = jax.random.normal(jax.random.key(42), (N, D_LATENT), jnp.float32)
    z, m, s = _tc_c(a3, scales, eps)
    return (z, m, s)


# speedup vs baseline: 7.7996x; 7.7996x over previous
import functools

import jax
import jax.numpy as jnp
from jax import lax
from jax.experimental import pallas as pl
from jax.experimental.pallas import tpu as pltpu
from jax.experimental.pallas import tpu_sc as plsc

N = 10000
E = 320000
D_FEAT = 128
D_H = 64
D_LATENT = 32
DW = 16

NC, NS = 2, 16
NW = NC * NS
EPW = E // NW
CH = 80
NCH = EPW // CH

N_PAD = 10240
RPT_DEG = 2 * N_PAD // NS
RPT = N_PAD // NS

_MESH = plsc.VectorSubcoreMesh(core_axis_name="c", subcore_axis_name="s",
                               num_cores=NC, num_subcores=NS)


@functools.partial(
    pl.kernel,
    out_type=jax.ShapeDtypeStruct((NC, 2 * N_PAD, DW), jnp.float32),
    mesh=_MESH,
    scratch_types=[
        pltpu.VMEM((NCH, CH), jnp.int32),
        pltpu.VMEM((NCH, CH), jnp.int32),
        pltpu.VMEM((CH, DW), jnp.float32),
        pltpu.VMEM_SHARED((2 * N_PAD, DW), jnp.float32),
    ],
    compiler_params=pltpu.CompilerParams(use_tc_tiling_on_sc=False),
)
def _deg_kernel(src_h, dstn_h, ones_h, zeros_h, out_h, srcv, dstv, onesv, acc):
    c = lax.axis_index("c")
    s = lax.axis_index("s")
    wid = s * NC + c
    pltpu.sync_copy(src_h.at[wid], srcv)
    pltpu.sync_copy(dstn_h.at[wid], dstv)
    pltpu.sync_copy(ones_h, onesv)
    pltpu.sync_copy(zeros_h.at[pl.ds(s * RPT_DEG, RPT_DEG)],
                    acc.at[pl.ds(s * RPT_DEG, RPT_DEG)])
    plsc.subcore_barrier()

    def body(j, carry):
        pltpu.sync_copy(onesv, acc.at[srcv.at[j]], add=True)
        pltpu.sync_copy(onesv, acc.at[dstv.at[j]], add=True)
        return carry

    lax.fori_loop(0, NCH, body, 0)
    plsc.subcore_barrier()
    pltpu.sync_copy(acc.at[pl.ds(s * RPT_DEG, RPT_DEG)],
                    out_h.at[c, pl.ds(s * RPT_DEG, RPT_DEG)])


@functools.partial(
    pl.kernel,
    out_type=jax.ShapeDtypeStruct((NC, N_PAD, D_H), jnp.float32),
    mesh=_MESH,
    scratch_types=[
        pltpu.VMEM((NCH, CH), jnp.int32),
        pltpu.VMEM((NCH, CH), jnp.int32),
        pltpu.VMEM((CH, D_H), jnp.float32),
        pltpu.VMEM_SHARED((N_PAD, D_H), jnp.float32),
        pltpu.SemaphoreType.DMA,
    ],
    compiler_params=pltpu.CompilerParams(use_tc_tiling_on_sc=False),
)
def _agg_kernel(src_h, dst_h, table_h, zeros_h, out_h, srcv, dstv, rows, acc, sem):
    c = lax.axis_index("c")
    s = lax.axis_index("s")
    wid = s * NC + c
    pltpu.sync_copy(src_h.at[wid], srcv)
    pltpu.sync_copy(dst_h.at[wid], dstv)
    pltpu.sync_copy(zeros_h.at[pl.ds(s * RPT, RPT)], acc.at[pl.ds(s * RPT, RPT)])
    plsc.subcore_barrier()

    def body(j, carry):
        pltpu.async_copy(table_h.at[srcv.at[j]], rows, sem).wait()
        pltpu.sync_copy(rows, acc.at[dstv.at[j]], add=True)
        return carry

    lax.fori_loop(0, NCH, body, 0)
    plsc.subcore_barrier()
    pltpu.sync_copy(acc.at[pl.ds(s * RPT, RPT)], out_h.at[c, pl.ds(s * RPT, RPT)])


BR = 1000
GRID = N // BR


def _tc_a_body(degp_ref, x_ref, w_ref, t_ref, scales_ref):
    dp = degp_ref[...]
    deg = jnp.maximum(jnp.sum(dp, axis=(0, 3)) * (1.0 / DW), 1.0)
    sc = lax.rsqrt(deg)
    scales_ref[...] = sc[None]
    t_ref[...] = jnp.dot(x_ref[...], w_ref[...],
                         preferred_element_type=jnp.float32) * sc[0][:, None]


def _tc_a(degp, x, W1):
    return pl.pallas_call(
        _tc_a_body,
        grid=(GRID,),
        in_specs=[
            pl.BlockSpec((NC, 2, BR, DW), lambda i: (0, 0, i, 0)),
            pl.BlockSpec((BR, D_FEAT), lambda i: (i, 0)),
            pl.BlockSpec((D_FEAT, D_H), lambda i: (0, 0)),
        ],
        out_specs=[
            pl.BlockSpec((BR, D_H), lambda i: (i, 0)),
            pl.BlockSpec((1, 2, BR), lambda i: (i, 0, 0)),
        ],
        out_shape=[
            jax.ShapeDtypeStruct((N, D_H), jnp.float32),
            jax.ShapeDtypeStruct((GRID, 2, BR), jnp.float32),
        ],
    )(degp, x, W1)


def _tc_b_body(aggp_ref, scales_ref, w_ref, t_ref):
    sc = scales_ref[0]
    a = aggp_ref[...]
    h = jnp.tanh((a[0] + a[1]) * sc[1][:, None])
    t_ref[...] = jnp.dot(h, w_ref[...],
                         preferred_element_type=jnp.float32) * sc[0][:, None]


def _tc_b(aggp, scales, W):
    return pl.pallas_call(
        _tc_b_body,
        grid=(GRID,),
        in_specs=[
            pl.BlockSpec((NC, BR, D_H), lambda i: (0, i, 0)),
            pl.BlockSpec((1, 2, BR), lambda i: (i, 0, 0)),
            pl.BlockSpec((D_H, D_H), lambda i: (0, 0)),
        ],
        out_specs=pl.BlockSpec((BR, D_H), lambda i: (i, 0)),
        out_shape=jax.ShapeDtypeStruct((N, D_H), jnp.float32),
    )(aggp, scales, W)


def _tc_c_body(aggp_ref, scales_ref, eps_ref, z_ref, m_ref, s_ref):
    sc = scales_ref[0]
    a = aggp_ref[...]
    ms = (a[0] + a[1]) * sc[1][:, None]
    m = ms[:, :D_LATENT]
    std = jnp.maximum(ms[:, D_LATENT:], 0.0) + 0.0001
    m_ref[...] = m
    s_ref[...] = std
    z_ref[...] = eps_ref[...] * std + m


def _tc_c(aggp, scales, eps):
    return pl.pallas_call(
        _tc_c_body,
        grid=(GRID,),
        in_specs=[
            pl.BlockSpec((NC, BR, D_H), lambda i: (0, i, 0)),
            pl.BlockSpec((1, 2, BR), lambda i: (i, 0, 0)),
            pl.BlockSpec((BR, D_LATENT), lambda i: (i, 0)),
        ],
        out_specs=[
            pl.BlockSpec((BR, D_LATENT), lambda i: (i, 0)),
            pl.BlockSpec((BR, D_LATENT), lambda i: (i, 0)),
            pl.BlockSpec((BR, D_LATENT), lambda i: (i, 0)),
        ],
        out_shape=[
            jax.ShapeDtypeStruct((N, D_LATENT), jnp.float32),
            jax.ShapeDtypeStruct((N, D_LATENT), jnp.float32),
            jax.ShapeDtypeStruct((N, D_LATENT), jnp.float32),
        ],
    )(aggp, scales, eps)


_DBG_JAX_DEG = False
_DBG_JAX_AGG = False


def _jax_degp(src, dst):
    deg_o = jnp.bincount(src, length=N).astype(jnp.float32)
    deg_i = jnp.bincount(dst, length=N).astype(jnp.float32)
    degp = jnp.zeros((NC, 2, N_PAD, DW), jnp.float32)
    degp = degp.at[0, 0, :N, 0].set(deg_o).at[0, 1, :N, 0].set(deg_i)
    return degp


def _jax_agg(src, dst, t):
    a = jax.ops.segment_sum(t[src], dst, num_segments=N)
    a = jnp.pad(a, ((0, N_PAD - N), (0, 0)))
    return jnp.stack([a, jnp.zeros_like(a)])


def kernel(adj, x, W1, W2, Wm, Ws):
    src = adj[0].astype(jnp.int32)
    dst = adj[1].astype(jnp.int32)
    src3 = src.reshape(NW, NCH, CH)
    dst3 = dst.reshape(NW, NCH, CH)
    dstn3 = (dst + N_PAD).reshape(NW, NCH, CH)
    ones_rows = jnp.ones((CH, DW), jnp.float32)
    zeros_deg = jnp.zeros((2 * N_PAD, DW), jnp.float32)
    zeros_acc = jnp.zeros((N_PAD, D_H), jnp.float32)

    if _DBG_JAX_DEG:
        degp = _jax_degp(src, dst)
    else:
        degp = _deg_kernel(src3, dstn3, ones_rows, zeros_deg)
        degp = degp.reshape(NC, 2, N_PAD, DW)
    t1, scales = _tc_a(degp, x, W1)

    def _agg(t):
        if _DBG_JAX_AGG:
            return _jax_agg(src, dst, t)
        return _agg_kernel(src3, dst3, t, zeros_acc)

    a1 = _agg(t1)
    t2 = _tc_b(a1, scales, W2)

    a2 = _agg(t2)
    W34 = jnp.concatenate([Wm, Ws], axis=1)
    t3 = _tc_b(a2, scales, W34)

    a3 = _agg(t3)
    eps = jax.random.normal(jax.random.key(42), (N, D_LATENT), jnp.float32)
    z, m, s = _tc_c(a3, scales, eps)
    return (z, m, s)


# speedup vs baseline: 11.5227x; 1.4773x over previous
import functools

import jax
import jax.numpy as jnp
from jax import lax
from jax.experimental import pallas as pl
from jax.experimental.pallas import tpu as pltpu
from jax.experimental.pallas import tpu_sc as plsc

N = 10000
E = 320000
D_FEAT = 128
D_H = 64
D_LATENT = 32
DW = 16

NC, NS = 2, 16
NW = NC * NS
EPW = E // NW
CH = 80
NCH = EPW // CH

N_PAD = 10240
RPT_DEG = 2 * N_PAD // NS
RPT = N_PAD // NS

_MESH = plsc.VectorSubcoreMesh(core_axis_name="c", subcore_axis_name="s",
                               num_cores=NC, num_subcores=NS)


@functools.partial(
    pl.kernel,
    out_type=jax.ShapeDtypeStruct((NC, 2 * N_PAD, DW), jnp.float32),
    mesh=_MESH,
    scratch_types=[
        pltpu.VMEM((NCH, CH), jnp.int32),
        pltpu.VMEM((NCH, CH), jnp.int32),
        pltpu.VMEM((CH, DW), jnp.float32),
        pltpu.VMEM_SHARED((2 * N_PAD, DW), jnp.float32),
        pltpu.SemaphoreType.DMA,
    ],
    compiler_params=pltpu.CompilerParams(use_tc_tiling_on_sc=False),
)
def _deg_kernel(src_h, dstn_h, ones_h, zeros_h, out_h, srcv, dstv, onesv, acc, sem):
    c = lax.axis_index("c")
    s = lax.axis_index("s")
    wid = s * NC + c
    pltpu.sync_copy(src_h.at[wid], srcv)
    pltpu.sync_copy(dstn_h.at[wid], dstv)
    pltpu.sync_copy(ones_h, onesv)
    pltpu.sync_copy(zeros_h.at[pl.ds(s * RPT_DEG, RPT_DEG)],
                    acc.at[pl.ds(s * RPT_DEG, RPT_DEG)])
    plsc.subcore_barrier()

    def body(j, carry):
        pltpu.async_copy(onesv, acc.at[srcv.at[j]], sem, add=True)
        pltpu.async_copy(onesv, acc.at[dstv.at[j]], sem, add=True)

        @pl.when(j > 0)
        def _():
            pltpu.make_async_copy(onesv, acc.at[srcv.at[j]], sem).wait()
            pltpu.make_async_copy(onesv, acc.at[srcv.at[j]], sem).wait()

        return carry

    lax.fori_loop(0, NCH, body, 0)
    pltpu.make_async_copy(onesv, acc.at[srcv.at[0]], sem).wait()
    pltpu.make_async_copy(onesv, acc.at[srcv.at[0]], sem).wait()
    plsc.subcore_barrier()
    pltpu.sync_copy(acc.at[pl.ds(s * RPT_DEG, RPT_DEG)],
                    out_h.at[c, pl.ds(s * RPT_DEG, RPT_DEG)])


@functools.partial(
    pl.kernel,
    out_type=jax.ShapeDtypeStruct((NC, N_PAD, D_H), jnp.float32),
    mesh=_MESH,
    scratch_types=[
        pltpu.VMEM((NCH, CH), jnp.int32),
        pltpu.VMEM((NCH, CH), jnp.int32),
        pltpu.VMEM((CH, D_H), jnp.float32),
        pltpu.VMEM((CH, D_H), jnp.float32),
        pltpu.VMEM_SHARED((N_PAD, D_H), jnp.float32),
        pltpu.SemaphoreType.DMA,
        pltpu.SemaphoreType.DMA,
        pltpu.SemaphoreType.DMA,
        pltpu.SemaphoreType.DMA,
    ],
    compiler_params=pltpu.CompilerParams(use_tc_tiling_on_sc=False),
)
def _agg_kernel(src_h, dst_h, table_h, zeros_h, out_h,
                srcv, dstv, rows0, rows1, acc, gs0, gs1, ss0, ss1):
    c = lax.axis_index("c")
    s = lax.axis_index("s")
    wid = s * NC + c
    pltpu.sync_copy(src_h.at[wid], srcv)
    pltpu.sync_copy(dst_h.at[wid], dstv)
    pltpu.sync_copy(zeros_h.at[pl.ds(s * RPT, RPT)], acc.at[pl.ds(s * RPT, RPT)])
    plsc.subcore_barrier()

    def gather(j, rows, sem):
        return pltpu.async_copy(table_h.at[srcv.at[j]], rows, sem)

    def scatter(j, rows, sem):
        return pltpu.async_copy(rows, acc.at[dstv.at[j]], sem, add=True)

    gather(0, rows0, gs0)

    def body(i, carry):
        j0 = i * 2
        gather(j0 + 1, rows1, gs1)
        pltpu.make_async_copy(table_h.at[srcv.at[j0]], rows0, gs0).wait()
        scatter(j0, rows0, ss0)
        pltpu.make_async_copy(rows0, acc.at[dstv.at[j0]], ss0).wait()
        gather(j0 + 2, rows0, gs0)
        pltpu.make_async_copy(table_h.at[srcv.at[j0 + 1]], rows1, gs1).wait()
        scatter(j0 + 1, rows1, ss1)
        pltpu.make_async_copy(rows1, acc.at[dstv.at[j0 + 1]], ss1).wait()
        return carry

    lax.fori_loop(0, (NCH - 1) // 2, body, 0)
    pltpu.make_async_copy(table_h.at[srcv.at[NCH - 1]], rows0, gs0).wait()
    scatter(NCH - 1, rows0, ss0)
    pltpu.make_async_copy(rows0, acc.at[dstv.at[NCH - 1]], ss0).wait()
    plsc.subcore_barrier()
    pltpu.sync_copy(acc.at[pl.ds(s * RPT, RPT)], out_h.at[c, pl.ds(s * RPT, RPT)])


BR = 1000
GRID = N // BR


def _tc_a_body(degp_ref, x_ref, w_ref, t_ref, scales_ref):
    dp = degp_ref[...]
    deg = jnp.maximum(jnp.sum(dp, axis=(0, 3)) * (1.0 / DW), 1.0)
    sc = lax.rsqrt(deg)
    scales_ref[...] = sc[None]
    t_ref[...] = jnp.dot(x_ref[...], w_ref[...],
                         preferred_element_type=jnp.float32) * sc[0][:, None]


def _tc_a(degp, x, W1):
    return pl.pallas_call(
        _tc_a_body,
        grid=(GRID,),
        in_specs=[
            pl.BlockSpec((NC, 2, BR, DW), lambda i: (0, 0, i, 0)),
            pl.BlockSpec((BR, D_FEAT), lambda i: (i, 0)),
            pl.BlockSpec((D_FEAT, D_H), lambda i: (0, 0)),
        ],
        out_specs=[
            pl.BlockSpec((BR, D_H), lambda i: (i, 0)),
            pl.BlockSpec((1, 2, BR), lambda i: (i, 0, 0)),
        ],
        out_shape=[
            jax.ShapeDtypeStruct((N, D_H), jnp.float32),
            jax.ShapeDtypeStruct((GRID, 2, BR), jnp.float32),
        ],
    )(degp, x, W1)


def _tc_b_body(aggp_ref, scales_ref, w_ref, t_ref):
    sc = scales_ref[0]
    a = aggp_ref[...]
    h = jnp.tanh((a[0] + a[1]) * sc[1][:, None])
    t_ref[...] = jnp.dot(h, w_ref[...],
                         preferred_element_type=jnp.float32) * sc[0][:, None]


def _tc_b(aggp, scales, W):
    return pl.pallas_call(
        _tc_b_body,
        grid=(GRID,),
        in_specs=[
            pl.BlockSpec((NC, BR, D_H), lambda i: (0, i, 0)),
            pl.BlockSpec((1, 2, BR), lambda i: (i, 0, 0)),
            pl.BlockSpec((D_H, D_H), lambda i: (0, 0)),
        ],
        out_specs=pl.BlockSpec((BR, D_H), lambda i: (i, 0)),
        out_shape=jax.ShapeDtypeStruct((N, D_H), jnp.float32),
    )(aggp, scales, W)


def _tc_c_body(aggp_ref, scales_ref, eps_ref, z_ref, m_ref, s_ref):
    sc = scales_ref[0]
    a = aggp_ref[...]
    ms = (a[0] + a[1]) * sc[1][:, None]
    m = ms[:, :D_LATENT]
    std = jnp.maximum(ms[:, D_LATENT:], 0.0) + 0.0001
    m_ref[...] = m
    s_ref[...] = std
    z_ref[...] = eps_ref[...] * std + m


def _tc_c(aggp, scales, eps):
    return pl.pallas_call(
        _tc_c_body,
        grid=(GRID,),
        in_specs=[
            pl.BlockSpec((NC, BR, D_H), lambda i: (0, i, 0)),
            pl.BlockSpec((1, 2, BR), lambda i: (i, 0, 0)),
            pl.BlockSpec((BR, D_LATENT), lambda i: (i, 0)),
        ],
        out_specs=[
            pl.BlockSpec((BR, D_LATENT), lambda i: (i, 0)),
            pl.BlockSpec((BR, D_LATENT), lambda i: (i, 0)),
            pl.BlockSpec((BR, D_LATENT), lambda i: (i, 0)),
        ],
        out_shape=[
            jax.ShapeDtypeStruct((N, D_LATENT), jnp.float32),
            jax.ShapeDtypeStruct((N, D_LATENT), jnp.float32),
            jax.ShapeDtypeStruct((N, D_LATENT), jnp.float32),
        ],
    )(aggp, scales, eps)


_DBG_JAX_DEG = False
_DBG_JAX_AGG = False


def _jax_degp(src, dst):
    deg_o = jnp.bincount(src, length=N).astype(jnp.float32)
    deg_i = jnp.bincount(dst, length=N).astype(jnp.float32)
    degp = jnp.zeros((NC, 2, N_PAD, DW), jnp.float32)
    degp = degp.at[0, 0, :N, 0].set(deg_o).at[0, 1, :N, 0].set(deg_i)
    return degp


def _jax_agg(src, dst, t):
    a = jax.ops.segment_sum(t[src], dst, num_segments=N)
    a = jnp.pad(a, ((0, N_PAD - N), (0, 0)))
    return jnp.stack([a, jnp.zeros_like(a)])


def kernel(adj, x, W1, W2, Wm, Ws):
    src = adj[0].astype(jnp.int32)
    dst = adj[1].astype(jnp.int32)
    src3 = src.reshape(NW, NCH, CH)
    dst3 = dst.reshape(NW, NCH, CH)
    dstn3 = (dst + N_PAD).reshape(NW, NCH, CH)
    ones_rows = jnp.ones((CH, DW), jnp.float32)
    zeros_deg = jnp.zeros((2 * N_PAD, DW), jnp.float32)
    zeros_acc = jnp.zeros((N_PAD, D_H), jnp.float32)

    if _DBG_JAX_DEG:
        degp = _jax_degp(src, dst)
    else:
        degp = _deg_kernel(src3, dstn3, ones_rows, zeros_deg)
        degp = degp.reshape(NC, 2, N_PAD, DW)
    t1, scales = _tc_a(degp, x, W1)

    def _agg(t):
        if _DBG_JAX_AGG:
            return _jax_agg(src, dst, t)
        return _agg_kernel(src3, dst3, t, zeros_acc)

    a1 = _agg(t1)
    t2 = _tc_b(a1, scales, W2)

    a2 = _agg(t2)
    W34 = jnp.concatenate([Wm, Ws], axis=1)
    t3 = _tc_b(a2, scales, W34)

    a3 = _agg(t3)
    eps = jax.random.normal(jax.random.key(42), (N, D_LATENT), jnp.float32)
    z, m, s = _tc_c(a3, scales, eps)
    return (z, m, s)


# speedup vs baseline: 13.8174x; 1.1991x over previous
import functools

import jax
import jax.numpy as jnp
from jax import lax
from jax.experimental import pallas as pl
from jax.experimental.pallas import tpu as pltpu
from jax.experimental.pallas import tpu_sc as plsc

N = 10000
E = 320000
D_FEAT = 128
D_H = 64
D_LATENT = 32
DW = 16

NC, NS = 2, 16
NW = NC * NS
EPW = E // NW
CH = 80
NCH = EPW // CH

N_PAD = 10240
RPT_DEG = 2 * N_PAD // NS
RPT = N_PAD // NS

_MESH = plsc.VectorSubcoreMesh(core_axis_name="c", subcore_axis_name="s",
                               num_cores=NC, num_subcores=NS)


@functools.partial(
    pl.kernel,
    out_type=jax.ShapeDtypeStruct((NC, 2 * N_PAD, DW), jnp.float32),
    mesh=_MESH,
    scratch_types=[
        pltpu.VMEM((NCH, CH), jnp.int32),
        pltpu.VMEM((NCH, CH), jnp.int32),
        pltpu.VMEM((CH, DW), jnp.float32),
        pltpu.VMEM_SHARED((2 * N_PAD, DW), jnp.float32),
        pltpu.SemaphoreType.DMA,
    ],
    compiler_params=pltpu.CompilerParams(use_tc_tiling_on_sc=False),
)
def _deg_kernel(src_h, dstn_h, ones_h, zeros_h, out_h, srcv, dstv, onesv, acc, sem):
    c = lax.axis_index("c")
    s = lax.axis_index("s")
    wid = s * NC + c
    pltpu.sync_copy(src_h.at[wid], srcv)
    pltpu.sync_copy(dstn_h.at[wid], dstv)
    pltpu.sync_copy(ones_h, onesv)
    pltpu.sync_copy(zeros_h.at[pl.ds(s * RPT_DEG, RPT_DEG)],
                    acc.at[pl.ds(s * RPT_DEG, RPT_DEG)])
    plsc.subcore_barrier()

    def body(j, carry):
        pltpu.async_copy(onesv, acc.at[srcv.at[j]], sem, add=True)
        pltpu.async_copy(onesv, acc.at[dstv.at[j]], sem, add=True)

        @pl.when(j > 0)
        def _():
            pltpu.make_async_copy(onesv, acc.at[srcv.at[j]], sem).wait()
            pltpu.make_async_copy(onesv, acc.at[srcv.at[j]], sem).wait()

        return carry

    lax.fori_loop(0, NCH, body, 0)
    pltpu.make_async_copy(onesv, acc.at[srcv.at[0]], sem).wait()
    pltpu.make_async_copy(onesv, acc.at[srcv.at[0]], sem).wait()
    plsc.subcore_barrier()
    pltpu.sync_copy(acc.at[pl.ds(s * RPT_DEG, RPT_DEG)],
                    out_h.at[c, pl.ds(s * RPT_DEG, RPT_DEG)])


@functools.partial(
    pl.kernel,
    out_type=jax.ShapeDtypeStruct((NC, N_PAD, D_H), jnp.float32),
    mesh=_MESH,
    scratch_types=[
        pltpu.VMEM((NCH, CH), jnp.int32),
        pltpu.VMEM((NCH, CH), jnp.int32),
        pltpu.VMEM((CH, D_H), jnp.float32),
        pltpu.VMEM((CH, D_H), jnp.float32),
        pltpu.VMEM((CH, D_H), jnp.float32),
        pltpu.VMEM((CH, D_H), jnp.float32),
        pltpu.VMEM_SHARED((N_PAD, D_H), jnp.float32),
        pltpu.SemaphoreType.DMA,
        pltpu.SemaphoreType.DMA,
        pltpu.SemaphoreType.DMA,
        pltpu.SemaphoreType.DMA,
        pltpu.SemaphoreType.DMA,
        pltpu.SemaphoreType.DMA,
        pltpu.SemaphoreType.DMA,
        pltpu.SemaphoreType.DMA,
    ],
    compiler_params=pltpu.CompilerParams(use_tc_tiling_on_sc=False),
)
def _agg_kernel(src_h, dst_h, table_h, zeros_h, out_h,
                srcv, dstv, r0, r1, r2, r3, acc,
                g0, g1, g2, g3, s0, s1, s2, s3):
    c = lax.axis_index("c")
    s = lax.axis_index("s")
    wid = s * NC + c
    bufs = (r0, r1, r2, r3)
    gsem = (g0, g1, g2, g3)
    ssem = (s0, s1, s2, s3)
    pltpu.sync_copy(src_h.at[wid], srcv)
    pltpu.sync_copy(dst_h.at[wid], dstv)
    pltpu.sync_copy(zeros_h.at[pl.ds(s * RPT, RPT)], acc.at[pl.ds(s * RPT, RPT)])
    plsc.subcore_barrier()

    def G(j, b):
        pltpu.async_copy(table_h.at[srcv.at[j]], bufs[b], gsem[b])

    def Gw(j, b):
        pltpu.make_async_copy(table_h.at[srcv.at[j]], bufs[b], gsem[b]).wait()

    def S(j, b):
        pltpu.async_copy(bufs[b], acc.at[dstv.at[j]], ssem[b], add=True)

    def Sw(j, b):
        pltpu.make_async_copy(bufs[b], acc.at[dstv.at[j]], ssem[b]).wait()

    G(0, 0)
    G(1, 1)
    G(2, 2)
    Gw(0, 0)
    S(0, 0)
    G(3, 3)
    Gw(1, 1)
    S(1, 1)

    def body(i, carry):
        j0 = 2 + i * 4
        for u in range(4):
            j = j0 + u
            b = (2 + u) % 4
            bb = (b + 2) % 4
            Sw(j - 2, bb)
            G(j + 2, bb)
            Gw(j, b)
            S(j, b)
        return carry

    lax.fori_loop(0, 30, body, 0)
    Sw(120, 0)
    G(124, 0)
    Gw(122, 2)
    S(122, 2)
    Gw(123, 3)
    S(123, 3)
    Gw(124, 0)
    S(124, 0)
    Sw(121, 1)
    Sw(122, 2)
    Sw(123, 3)
    Sw(124, 0)
    plsc.subcore_barrier()
    pltpu.sync_copy(acc.at[pl.ds(s * RPT, RPT)], out_h.at[c, pl.ds(s * RPT, RPT)])


BR = 1000
GRID = N // BR


def _tc_a_body(degp_ref, x_ref, w_ref, t_ref, scales_ref):
    dp = degp_ref[...]
    deg = jnp.maximum(jnp.sum(dp, axis=(0, 3)) * (1.0 / DW), 1.0)
    sc = lax.rsqrt(deg)
    scales_ref[...] = sc[None]
    t_ref[...] = jnp.dot(x_ref[...], w_ref[...],
                         preferred_element_type=jnp.float32) * sc[0][:, None]


def _tc_a(degp, x, W1):
    return pl.pallas_call(
        _tc_a_body,
        grid=(GRID,),
        in_specs=[
            pl.BlockSpec((NC, 2, BR, DW), lambda i: (0, 0, i, 0)),
            pl.BlockSpec((BR, D_FEAT), lambda i: (i, 0)),
            pl.BlockSpec((D_FEAT, D_H), lambda i: (0, 0)),
        ],
        out_specs=[
            pl.BlockSpec((BR, D_H), lambda i: (i, 0)),
            pl.BlockSpec((1, 2, BR), lambda i: (i, 0, 0)),
        ],
        out_shape=[
            jax.ShapeDtypeStruct((N, D_H), jnp.float32),
            jax.ShapeDtypeStruct((GRID, 2, BR), jnp.float32),
        ],
    )(degp, x, W1)


def _tc_b_body(aggp_ref, scales_ref, w_ref, t_ref):
    sc = scales_ref[0]
    a = aggp_ref[...]
    h = jnp.tanh((a[0] + a[1]) * sc[1][:, None])
    t_ref[...] = jnp.dot(h, w_ref[...],
                         preferred_element_type=jnp.float32) * sc[0][:, None]


def _tc_b(aggp, scales, W):
    return pl.pallas_call(
        _tc_b_body,
        grid=(GRID,),
        in_specs=[
            pl.BlockSpec((NC, BR, D_H), lambda i: (0, i, 0)),
            pl.BlockSpec((1, 2, BR), lambda i: (i, 0, 0)),
            pl.BlockSpec((D_H, D_H), lambda i: (0, 0)),
        ],
        out_specs=pl.BlockSpec((BR, D_H), lambda i: (i, 0)),
        out_shape=jax.ShapeDtypeStruct((N, D_H), jnp.float32),
    )(aggp, scales, W)


def _tc_c_body(aggp_ref, scales_ref, eps_ref, z_ref, m_ref, s_ref):
    sc = scales_ref[0]
    a = aggp_ref[...]
    ms = (a[0] + a[1]) * sc[1][:, None]
    m = ms[:, :D_LATENT]
    std = jnp.maximum(ms[:, D_LATENT:], 0.0) + 0.0001
    m_ref[...] = m
    s_ref[...] = std
    z_ref[...] = eps_ref[...] * std + m


def _tc_c(aggp, scales, eps):
    return pl.pallas_call(
        _tc_c_body,
        grid=(GRID,),
        in_specs=[
            pl.BlockSpec((NC, BR, D_H), lambda i: (0, i, 0)),
            pl.BlockSpec((1, 2, BR), lambda i: (i, 0, 0)),
            pl.BlockSpec((BR, D_LATENT), lambda i: (i, 0)),
        ],
        out_specs=[
            pl.BlockSpec((BR, D_LATENT), lambda i: (i, 0)),
            pl.BlockSpec((BR, D_LATENT), lambda i: (i, 0)),
            pl.BlockSpec((BR, D_LATENT), lambda i: (i, 0)),
        ],
        out_shape=[
            jax.ShapeDtypeStruct((N, D_LATENT), jnp.float32),
            jax.ShapeDtypeStruct((N, D_LATENT), jnp.float32),
            jax.ShapeDtypeStruct((N, D_LATENT), jnp.float32),
        ],
    )(aggp, scales, eps)


_DBG_JAX_DEG = False
_DBG_JAX_AGG = False


def _jax_degp(src, dst):
    deg_o = jnp.bincount(src, length=N).astype(jnp.float32)
    deg_i = jnp.bincount(dst, length=N).astype(jnp.float32)
    degp = jnp.zeros((NC, 2, N_PAD, DW), jnp.float32)
    degp = degp.at[0, 0, :N, 0].set(deg_o).at[0, 1, :N, 0].set(deg_i)
    return degp


def _jax_agg(src, dst, t):
    a = jax.ops.segment_sum(t[src], dst, num_segments=N)
    a = jnp.pad(a, ((0, N_PAD - N), (0, 0)))
    return jnp.stack([a, jnp.zeros_like(a)])


def kernel(adj, x, W1, W2, Wm, Ws):
    src = adj[0].astype(jnp.int32)
    dst = adj[1].astype(jnp.int32)
    src3 = src.reshape(NW, NCH, CH)
    dst3 = dst.reshape(NW, NCH, CH)
    dstn3 = (dst + N_PAD).reshape(NW, NCH, CH)
    ones_rows = jnp.ones((CH, DW), jnp.float32)
    zeros_deg = jnp.zeros((2 * N_PAD, DW), jnp.float32)
    zeros_acc = jnp.zeros((N_PAD, D_H), jnp.float32)

    if _DBG_JAX_DEG:
        degp = _jax_degp(src, dst)
    else:
        degp = _deg_kernel(src3, dstn3, ones_rows, zeros_deg)
        degp = degp.reshape(NC, 2, N_PAD, DW)
    t1, scales = _tc_a(degp, x, W1)

    def _agg(t):
        if _DBG_JAX_AGG:
            return _jax_agg(src, dst, t)
        return _agg_kernel(src3, dst3, t, zeros_acc)

    a1 = _agg(t1)
    t2 = _tc_b(a1, scales, W2)

    a2 = _agg(t2)
    W34 = jnp.concatenate([Wm, Ws], axis=1)
    t3 = _tc_b(a2, scales, W34)

    a3 = _agg(t3)
    eps = jax.random.normal(jax.random.key(42), (N, D_LATENT), jnp.float32)
    z, m, s = _tc_c(a3, scales, eps)
    return (z, m, s)
